# Initial kernel scaffold; baseline (speedup 1.0000x reference)
import jax
import jax.numpy as jnp
from jax.experimental import pallas as pl


def kernel(timepoint_ids, condition_ids, region_ids, timepoint_weight, condition_weight, region_weight):
    raise NotImplementedError("write your pallas kernel here")


# speedup vs baseline: 4.4415x; 4.4415x over previous
import functools

import jax
import jax.numpy as jnp
from jax import lax
from jax.experimental import pallas as pl
from jax.experimental.pallas import tpu as pltpu
from jax.experimental.pallas import tpu_sc as plsc

_NUM_CORES = 2
_NUM_SUBCORES = 16
_LANES = 16
_NUM_WORKERS = _NUM_CORES * _NUM_SUBCORES

_B = 16384
_D_T, _D_C, _D_R = 16, 8, 8
_D_OUT = _D_T + _D_C + _D_R
_B_PER_W = _B // _NUM_WORKERS
_GROUPS = _B_PER_W // _LANES


def _body(tid_hbm, cid_hbm, rid_hbm, tw_hbm, cw_hbm, rw_hbm, out_hbm,
          tid_v, cid_v, rid_v, tw_v, cw_v, rw_v, out_v):
    wid = lax.axis_index("s") * _NUM_CORES + lax.axis_index("c")
    base = wid * _B_PER_W

    pltpu.sync_copy(tid_hbm.at[pl.ds(base, _B_PER_W)], tid_v)
    pltpu.sync_copy(cid_hbm.at[pl.ds(base, _B_PER_W)], cid_v)
    pltpu.sync_copy(rid_hbm.at[pl.ds(base, _B_PER_W)], rid_v)
    pltpu.sync_copy(tw_hbm, tw_v)
    pltpu.sync_copy(cw_hbm, cw_v)
    pltpu.sync_copy(rw_hbm, rw_v)

    lanes = lax.iota(jnp.int32, _LANES)

    def group(g, _):
        row = g * _LANES + lanes
        tv = tid_v[pl.ds(g * _LANES, _LANES)]
        cv = cid_v[pl.ds(g * _LANES, _LANES)]
        rv = rid_v[pl.ds(g * _LANES, _LANES)]
        for j in range(_D_T):
            col = jnp.full((_LANES,), j, jnp.int32)
            plsc.store_scatter(out_v, [row, col], plsc.load_gather(tw_v, [tv, col]))
        for j in range(_D_C):
            col = jnp.full((_LANES,), j, jnp.int32)
            x = plsc.load_gather(cw_v, [cv, col])
            plsc.store_scatter(out_v, [row, col + _D_T], x)
        for j in range(_D_R):
            col = jnp.full((_LANES,), j, jnp.int32)
            x = plsc.load_gather(rw_v, [rv, col])
            plsc.store_scatter(out_v, [row, col + _D_T + _D_C], x)
        return 0

    lax.fori_loop(0, _GROUPS, group, 0)

    pltpu.sync_copy(out_v, out_hbm.at[pl.ds(base, _B_PER_W)])


@jax.jit
def _run(tid, cid, rid, tw, cw, rw):
    mesh = plsc.VectorSubcoreMesh(core_axis_name="c", subcore_axis_name="s")
    return pl.kernel(
        _body,
        out_type=jax.ShapeDtypeStruct((_B, _D_OUT), jnp.float32),
        mesh=mesh,
        compiler_params=pltpu.CompilerParams(needs_layout_passes=False),
        scratch_types=[
            pltpu.VMEM((_B_PER_W,), jnp.int32),
            pltpu.VMEM((_B_PER_W,), jnp.int32),
            pltpu.VMEM((_B_PER_W,), jnp.int32),
            pltpu.VMEM((5, _D_T), jnp.float32),
            pltpu.VMEM((2, _D_C), jnp.float32),
            pltpu.VMEM((2, _D_R), jnp.float32),
            pltpu.VMEM((_B_PER_W, _D_OUT), jnp.float32),
        ],
    )(tid, cid, rid, tw, cw, rw)


def kernel(timepoint_ids, condition_ids, region_ids, timepoint_weight,
           condition_weight, region_weight):
    return _run(
        jnp.asarray(timepoint_ids, jnp.int32),
        jnp.asarray(condition_ids, jnp.int32),
        jnp.asarray(region_ids, jnp.int32),
        timepoint_weight,
        condition_weight,
        region_weight,
    )


# speedup vs baseline: 6.8479x; 1.5418x over previous
import jax
import jax.numpy as jnp
from jax import lax
from jax.experimental import pallas as pl
from jax.experimental.pallas import tpu as pltpu
from jax.experimental.pallas import tpu_sc as plsc

_NUM_CORES = 2
_NUM_SUBCORES = 16
_LANES = 16
_NUM_WORKERS = _NUM_CORES * _NUM_SUBCORES

_B = 16384
_D_T, _D_C, _D_R = 16, 8, 8
_D_OUT = _D_T + _D_C + _D_R
_N_COMB = 5 * 2 * 2
_B_PER_W = _B // _NUM_WORKERS
_GROUPS = _B_PER_W // _LANES
_CHUNK = 128


def _body(tid_hbm, cid_hbm, rid_hbm, tw_hbm, cw_hbm, rw_hbm, out_hbm,
          tid_v, cid_v, rid_v, tw_v, cw_v, rw_v, comb_v, idx_v, rows_v,
          comb_sh, sem):
    cid_ax = lax.axis_index("c")
    sid_ax = lax.axis_index("s")
    wid = sid_ax * _NUM_CORES + cid_ax
    base = wid * _B_PER_W

    @pl.when(sid_ax == 0)
    def _build():
        pltpu.sync_copy(tw_hbm, tw_v)
        pltpu.sync_copy(cw_hbm, cw_v)
        pltpu.sync_copy(rw_hbm, rw_v)
        lanes = lax.iota(jnp.int32, _LANES)
        for k in range(_N_COMB):
            ti, ci, ri = k // 4, (k // 2) % 2, k % 2
            comb_v[k, 0:_D_T] = tw_v[ti, :]
            idx_c = ci * _D_C + jnp.minimum(lanes, _D_C - 1)
            idx_r = ri * _D_R + jnp.maximum(lanes - _D_C, 0)
            cvals = plsc.load_gather(cw_v, [idx_c // _D_C, idx_c % _D_C])
            rvals = plsc.load_gather(rw_v, [idx_r // _D_R, idx_r % _D_R])
            comb_v[k, _D_T:_D_OUT] = jnp.where(lanes < _D_C, cvals, rvals)
        pltpu.sync_copy(comb_v, comb_sh)

    plsc.subcore_barrier()

    pltpu.sync_copy(tid_hbm.at[pl.ds(base, _B_PER_W)], tid_v)
    pltpu.sync_copy(cid_hbm.at[pl.ds(base, _B_PER_W)], cid_v)
    pltpu.sync_copy(rid_hbm.at[pl.ds(base, _B_PER_W)], rid_v)

    for g in range(_GROUPS):
        chunk, off = divmod(g * _LANES, _CHUNK)
        sl = pl.ds(g * _LANES, _LANES)
        idx_v[chunk, pl.ds(off, _LANES)] = (
            tid_v[sl] * 4 + cid_v[sl] * 2 + rid_v[sl])

    for ch in range(_B_PER_W // _CHUNK):
        pltpu.async_copy(comb_sh.at[idx_v.at[ch]],
                         rows_v.at[pl.ds(ch * _CHUNK, _CHUNK)], sem).wait()
    pltpu.sync_copy(rows_v, out_hbm.at[pl.ds(base, _B_PER_W)])


@jax.jit
def _run(tid, cid, rid, tw, cw, rw):
    mesh = plsc.VectorSubcoreMesh(core_axis_name="c", subcore_axis_name="s")
    return pl.kernel(
        _body,
        out_type=jax.ShapeDtypeStruct((_B, _D_OUT), jnp.float32),
        mesh=mesh,
        compiler_params=pltpu.CompilerParams(needs_layout_passes=False),
        scratch_types=[
            pltpu.VMEM((_B_PER_W,), jnp.int32),
            pltpu.VMEM((_B_PER_W,), jnp.int32),
            pltpu.VMEM((_B_PER_W,), jnp.int32),
            pltpu.VMEM((5, _D_T), jnp.float32),
            pltpu.VMEM((2, _D_C), jnp.float32),
            pltpu.VMEM((2, _D_R), jnp.float32),
            pltpu.VMEM((_N_COMB, _D_OUT), jnp.float32),
            pltpu.VMEM((_B_PER_W // _CHUNK, _CHUNK), jnp.int32),
            pltpu.VMEM((_B_PER_W, _D_OUT), jnp.float32),
            pltpu.VMEM_SHARED((_N_COMB, _D_OUT), jnp.float32),
            pltpu.SemaphoreType.DMA,
        ],
    )(tid, cid, rid, tw, cw, rw)


def kernel(timepoint_ids, condition_ids, region_ids, timepoint_weight,
           condition_weight, region_weight):
    return _run(
        jnp.asarray(timepoint_ids, jnp.int32),
        jnp.asarray(condition_ids, jnp.int32),
        jnp.asarray(region_ids, jnp.int32),
        timepoint_weight,
        condition_weight,
        region_weight,
    )


# speedup vs baseline: 6.9354x; 1.0128x over previous
import jax
import jax.numpy as jnp
from jax import lax
from jax.experimental import pallas as pl
from jax.experimental.pallas import tpu as pltpu
from jax.experimental.pallas import tpu_sc as plsc

_NUM_CORES = 2
_NUM_SUBCORES = 16
_LANES = 16
_NUM_WORKERS = _NUM_CORES * _NUM_SUBCORES

_B = 16384
_D_T, _D_C, _D_R = 16, 8, 8
_D_OUT = _D_T + _D_C + _D_R
_N_COMB = 5 * 2 * 2
_B_PER_W = _B // _NUM_WORKERS
_GROUPS = _B_PER_W // _LANES
_CHUNK = 128


def _body(tid_hbm, cid_hbm, rid_hbm, tw_hbm, cw_hbm, rw_hbm, out_hbm,
          tid_v, cid_v, rid_v, tw_v, cw_v, rw_v, comb_v, idx_v, rows_v,
          comb_sh, sem):
    cid_ax = lax.axis_index("c")
    sid_ax = lax.axis_index("s")
    wid = sid_ax * _NUM_CORES + cid_ax
    base = wid * _B_PER_W

    pltpu.sync_copy(tid_hbm.at[pl.ds(base, _B_PER_W)], tid_v)
    pltpu.sync_copy(cid_hbm.at[pl.ds(base, _B_PER_W)], cid_v)
    pltpu.sync_copy(rid_hbm.at[pl.ds(base, _B_PER_W)], rid_v)

    @pl.when(sid_ax == 0)
    def _build():
        pltpu.sync_copy(tw_hbm, tw_v)
        pltpu.sync_copy(cw_hbm, cw_v)
        pltpu.sync_copy(rw_hbm, rw_v)
        lanes = lax.iota(jnp.int32, _LANES)
        for k in range(_N_COMB):
            ti, ci, ri = k // 4, (k // 2) % 2, k % 2
            comb_v[k, 0:_D_T] = tw_v[ti, :]
            idx_c = ci * _D_C + jnp.minimum(lanes, _D_C - 1)
            idx_r = ri * _D_R + jnp.maximum(lanes - _D_C, 0)
            cvals = plsc.load_gather(cw_v, [idx_c // _D_C, idx_c % _D_C])
            rvals = plsc.load_gather(rw_v, [idx_r // _D_R, idx_r % _D_R])
            comb_v[k, _D_T:_D_OUT] = jnp.where(lanes < _D_C, cvals, rvals)
        pltpu.sync_copy(comb_v, comb_sh)

    plsc.subcore_barrier()

    for g in range(_GROUPS):
        chunk, off = divmod(g * _LANES, _CHUNK)
        sl = pl.ds(g * _LANES, _LANES)
        idx_v[chunk, pl.ds(off, _LANES)] = (
            tid_v[sl] * 4 + cid_v[sl] * 2 + rid_v[sl])

    descs = [
        pltpu.async_copy(comb_sh.at[idx_v.at[ch]],
                         rows_v.at[pl.ds(ch * _CHUNK, _CHUNK)], sem)
        for ch in range(_B_PER_W // _CHUNK)
    ]
    for d in descs:
        d.wait()
    pltpu.sync_copy(rows_v, out_hbm.at[pl.ds(base, _B_PER_W)])


@jax.jit
def _run(tid, cid, rid, tw, cw, rw):
    mesh = plsc.VectorSubcoreMesh(core_axis_name="c", subcore_axis_name="s",
                                  num_cores=_NUM_CORES)
    return pl.kernel(
        _body,
        out_type=jax.ShapeDtypeStruct((_B, _D_OUT), jnp.float32),
        mesh=mesh,
        compiler_params=pltpu.CompilerParams(needs_layout_passes=False),
        scratch_types=[
            pltpu.VMEM((_B_PER_W,), jnp.int32),
            pltpu.VMEM((_B_PER_W,), jnp.int32),
            pltpu.VMEM((_B_PER_W,), jnp.int32),
            pltpu.VMEM((5, _D_T), jnp.float32),
            pltpu.VMEM((2, _D_C), jnp.float32),
            pltpu.VMEM((2, _D_R), jnp.float32),
            pltpu.VMEM((_N_COMB, _D_OUT), jnp.float32),
            pltpu.VMEM((_B_PER_W // _CHUNK, _CHUNK), jnp.int32),
            pltpu.VMEM((_B_PER_W, _D_OUT), jnp.float32),
            pltpu.VMEM_SHARED((_N_COMB, _D_OUT), jnp.float32),
            pltpu.SemaphoreType.DMA,
        ],
    )(tid, cid, rid, tw, cw, rw)


def kernel(timepoint_ids, condition_ids, region_ids, timepoint_weight,
           condition_weight, region_weight):
    return _run(
        jnp.asarray(timepoint_ids, jnp.int32),
        jnp.asarray(condition_ids, jnp.int32),
        jnp.asarray(region_ids, jnp.int32),
        timepoint_weight,
        condition_weight,
        region_weight,
    )
